# Initial kernel scaffold; baseline (speedup 1.0000x reference)
import jax
import jax.numpy as jnp
from jax.experimental import pallas as pl


def kernel(x, edge_index, edge_label_index, W_t, b_t, W1, b1, W2, b2):
    raise NotImplementedError("write your pallas kernel here")


# speedup vs baseline: 6.4134x; 6.4134x over previous
import functools

import jax
import jax.numpy as jnp
from jax import lax
from jax.experimental import pallas as pl
from jax.experimental.pallas import tpu as pltpu
from jax.experimental.pallas import tpu_sc as plsc

N = 10000
E = 320000
EL = 320000
D = 128
F = 64
NC, NS, L = 2, 16, 16
ROWS_PT = 640
N_PAD = NS * ROWS_PT
TRASH = N
CH = 128
G = 8
EPT = 20480
NCH = EPT // CH
NG = NCH // G
E_PAD = EPT * NS
ELPT = 10240
NLCH = ELPT // CH
EL_PAD = ELPT * NS * NC

_mesh = plsc.VectorSubcoreMesh(
    core_axis_name="c", subcore_axis_name="s", num_cores=NC, num_subcores=NS)


def _tc_transform_body(x_ref, w_ref, b_ref, h_ref):
    h_ref[...] = jnp.maximum(
        jnp.dot(x_ref[...], w_ref[...], preferred_element_type=jnp.float32)
        + b_ref[...], 0.0)


def _tc_transform(x, wt_t, b_t):
    return pl.pallas_call(
        _tc_transform_body,
        grid=(10,),
        in_specs=[
            pl.BlockSpec((1000, D), lambda i: (i, 0)),
            pl.BlockSpec((D, D), lambda i: (0, 0)),
            pl.BlockSpec((1, D), lambda i: (0, 0)),
        ],
        out_specs=pl.BlockSpec((1000, D), lambda i: (i, 0)),
        out_shape=jax.ShapeDtypeStruct((N, D), jnp.float32),
    )(x, wt_t, b_t)


def _tc_proj_body(h_ref, wa_ref, wb_ref, b1_ref, a_ref, b_ref):
    hid = h_ref[...]
    n2 = jnp.sum(hid * hid, axis=1, keepdims=True)
    hn = hid / jnp.maximum(jnp.sqrt(n2), 1e-12)
    a_ref[...] = jnp.dot(hn, wa_ref[...],
                         preferred_element_type=jnp.float32) + b1_ref[...]
    b_ref[...] = jnp.dot(hn, wb_ref[...], preferred_element_type=jnp.float32)


def _tc_proj(hid, w1a_t, w1b_t, b1):
    return pl.pallas_call(
        _tc_proj_body,
        grid=(10,),
        in_specs=[
            pl.BlockSpec((1000, D), lambda i: (i, 0)),
            pl.BlockSpec((D, D), lambda i: (0, 0)),
            pl.BlockSpec((D, D), lambda i: (0, 0)),
            pl.BlockSpec((1, D), lambda i: (0, 0)),
        ],
        out_specs=[
            pl.BlockSpec((1000, D), lambda i: (i, 0)),
            pl.BlockSpec((1000, D), lambda i: (i, 0)),
        ],
        out_shape=[
            jax.ShapeDtypeStruct((N, D), jnp.float32),
            jax.ShapeDtypeStruct((N, D), jnp.float32),
        ],
    )(hid, w1a_t, w1b_t, b1)


def _sc_mp_body(h_hbm, row_hbm, col_hbm, out_hbm,
                table, acc, deg,
                rowi, coli, gbuf, hbuf, abuf, h2buf,
                degv, disv, ewv):
    cid = lax.axis_index("c")
    sid = lax.axis_index("s")
    rbase = sid * ROWS_PT

    zero16 = jnp.zeros((L,), jnp.float32)

    def zfill_body(r, _):
        for c4 in range(F // L):
            hbuf[r, pl.ds(c4 * L, L)] = zero16
        return _
    lax.fori_loop(0, 64, zfill_body, None)

    def zdeg_body(i, _):
        degv[pl.ds(i * L, L)] = zero16
        return _
    lax.fori_loop(0, ROWS_PT // L, zdeg_body, None)

    def zacc_body(k, _):
        pltpu.sync_copy(hbuf, acc.at[pl.ds(rbase + k * 64, 64), :])
        return _
    lax.fori_loop(0, ROWS_PT // 64, zacc_body, None)
    pltpu.sync_copy(degv, deg.at[pl.ds(rbase, ROWS_PT)])

    def hist_body(jj, _):
        pltpu.sync_copy(row_hbm.at[sid, pl.ds(jj * G, G), :], rowi)
        pltpu.sync_copy(col_hbm.at[sid, pl.ds(jj * G, G), :], coli)
        for j in range(G):
            for i in range(CH // L):
                r = rowi[j, pl.ds(i * L, L)]
                c = coli[j, pl.ds(i * L, L)]
                ewv[pl.ds(i * L, L)] = jnp.where(r == c, 0.0, 1.0)
            pltpu.sync_copy(ewv, deg.at[rowi.at[j]], add=True)
        return _
    lax.fori_loop(0, NG, hist_body, None)
    plsc.subcore_barrier()

    pltpu.sync_copy(deg.at[pl.ds(rbase, ROWS_PT)], degv)

    def dis_body(i, _):
        d = degv[pl.ds(i * L, L)] + 1.0
        z = 0.5 * (d + 1.0)
        for _n in range(15):
            z = 0.5 * (z + d / z)
        disv[pl.ds(i * L, L)] = 1.0 / z
        return _
    lax.fori_loop(0, ROWS_PT // L, dis_body, None)

    def scale_h_body(k, _):
        r0 = rbase + k * 64
        pltpu.sync_copy(h_hbm.at[cid, pl.ds(r0, 64), :], hbuf)

        def grp_body(g, _2):
            dis16 = disv[pl.ds(k * 64 + g * L, L)]
            for r16 in range(L):
                r = g * L + r16
                s = dis16[r16]
                for c4 in range(F // L):
                    hbuf[r, pl.ds(c4 * L, L)] = hbuf[r, pl.ds(c4 * L, L)] * s
            return _2
        lax.fori_loop(0, 64 // L, grp_body, None)
        pltpu.sync_copy(hbuf, table.at[pl.ds(r0, 64), :])
        return _
    lax.fori_loop(0, ROWS_PT // 64, scale_h_body, None)
    plsc.subcore_barrier()

    def edge_pass():
        def slab_body(jj, _):
            pltpu.sync_copy(row_hbm.at[sid, pl.ds(jj * G, G), :], rowi)
            pltpu.sync_copy(col_hbm.at[sid, pl.ds(jj * G, G), :], coli)
            for j in range(G):
                for i in range(CH // L):
                    r = rowi[j, pl.ds(i * L, L)]
                    c = coli[j, pl.ds(i * L, L)]
                    coli[j, pl.ds(i * L, L)] = jnp.where(r == c, TRASH, c)
                pltpu.sync_copy(table.at[rowi.at[j]], gbuf)
                pltpu.sync_copy(gbuf, acc.at[coli.at[j]], add=True)
            return _
        lax.fori_loop(0, NG, slab_body, None)
        plsc.subcore_barrier()

    edge_pass()

    def mid_body(k, _):
        r0 = rbase + k * 64
        pltpu.sync_copy(acc.at[pl.ds(r0, 64), :], abuf)
        pltpu.sync_copy(table.at[pl.ds(r0, 64), :], hbuf)

        def grp_body(g, _2):
            dis16 = disv[pl.ds(k * 64 + g * L, L)]
            for r16 in range(L):
                r = g * L + r16
                s = dis16[r16]
                for c4 in range(F // L):
                    sl = pl.ds(c4 * L, L)
                    s1 = (abuf[r, sl] + hbuf[r, sl]) * s
                    hbuf[r, sl] = s1 * s
                    abuf[r, sl] = zero16
            return _2
        lax.fori_loop(0, 64 // L, grp_body, None)
        pltpu.sync_copy(hbuf, table.at[pl.ds(r0, 64), :])
        pltpu.sync_copy(abuf, acc.at[pl.ds(r0, 64), :])
        return _
    lax.fori_loop(0, ROWS_PT // 64, mid_body, None)
    plsc.subcore_barrier()

    edge_pass()

    def fin_body(k, _):
        r0 = rbase + k * 64
        pltpu.sync_copy(acc.at[pl.ds(r0, 64), :], abuf)
        pltpu.sync_copy(table.at[pl.ds(r0, 64), :], hbuf)
        pltpu.sync_copy(h_hbm.at[cid, pl.ds(r0, 64), :], h2buf)

        def grp_body(g, _2):
            dis16 = disv[pl.ds(k * 64 + g * L, L)]
            inv16 = 1.0 / dis16
            for r16 in range(L):
                r = g * L + r16
                s = dis16[r16]
                inv = inv16[r16]
                for c4 in range(F // L):
                    sl = pl.ds(c4 * L, L)
                    t = hbuf[r, sl]
                    s2 = (abuf[r, sl] + t) * s
                    hbuf[r, sl] = h2buf[r, sl] + t * inv + s2
            return _2
        lax.fori_loop(0, 64 // L, grp_body, None)
        pltpu.sync_copy(hbuf, out_hbm.at[cid, pl.ds(r0, 64), :])
        return _
    lax.fori_loop(0, ROWS_PT // 64, fin_body, None)


_sc_mp = functools.partial(
    pl.kernel,
    out_type=jax.ShapeDtypeStruct((NC, N_PAD, F), jnp.float32),
    mesh=_mesh,
    scratch_types=[
        pltpu.VMEM_SHARED((N_PAD, F), jnp.float32),
        pltpu.VMEM_SHARED((N_PAD, F), jnp.float32),
        pltpu.VMEM_SHARED((N_PAD,), jnp.float32),
        pltpu.VMEM((G, CH), jnp.int32),
        pltpu.VMEM((G, CH), jnp.int32),
        pltpu.VMEM((CH, F), jnp.float32),
        pltpu.VMEM((64, F), jnp.float32),
        pltpu.VMEM((64, F), jnp.float32),
        pltpu.VMEM((64, F), jnp.float32),
        pltpu.VMEM((ROWS_PT,), jnp.float32),
        pltpu.VMEM((ROWS_PT,), jnp.float32),
        pltpu.VMEM((CH,), jnp.float32),
    ],
)(_sc_mp_body)


def _sc_head_body(a_hbm, b_hbm, li_hbm, lj_hbm, w_hbm, out_hbm,
                  liv, ljv, abuf, bbuf, wv, obuf):
    cid = lax.axis_index("c")
    sid = lax.axis_index("s")
    wid = sid * NC + cid
    pltpu.sync_copy(li_hbm.at[wid], liv)
    pltpu.sync_copy(lj_hbm.at[wid], ljv)
    pltpu.sync_copy(w_hbm, wv)
    w2c = [wv[pl.ds(c * L, L)] for c in range(D // L)]
    b2s = wv[pl.ds(D, L)][0]
    lane = lax.iota(jnp.int32, L)
    onehot = [jnp.where(lane == r, 1.0, 0.0) for r in range(L)]

    def chunk_body(j, _):
        pltpu.sync_copy(a_hbm.at[liv.at[j]], abuf)
        pltpu.sync_copy(b_hbm.at[ljv.at[j]], bbuf)

        def grp_body(g, _2):
            out16 = jnp.zeros((L,), jnp.float32) + b2s
            for e16 in range(L):
                e = g * L + e16
                acc16 = jnp.zeros((L,), jnp.float32)
                for c in range(D // L):
                    sl = pl.ds(c * L, L)
                    z = jnp.maximum(abuf[e, sl] + bbuf[e, sl], 0.0)
                    acc16 = acc16 + w2c[c] * z
                for sh in (8, 4, 2, 1):
                    acc16 = acc16 + acc16.at[lane ^ sh].get(
                        mode="promise_in_bounds")
                out16 = out16 + acc16 * onehot[e16]
            obuf[pl.ds(g * L, L)] = out16
            return _2
        lax.fori_loop(0, CH // L, grp_body, None)
        pltpu.sync_copy(obuf, out_hbm.at[pl.ds(wid * ELPT + j * CH, CH)])
        return _
    lax.fori_loop(0, NLCH, chunk_body, None)


_sc_head = functools.partial(
    pl.kernel,
    out_type=jax.ShapeDtypeStruct((EL_PAD,), jnp.float32),
    mesh=_mesh,
    scratch_types=[
        pltpu.VMEM((NLCH, CH), jnp.int32),
        pltpu.VMEM((NLCH, CH), jnp.int32),
        pltpu.VMEM((CH, D), jnp.float32),
        pltpu.VMEM((CH, D), jnp.float32),
        pltpu.VMEM((D + L,), jnp.float32),
        pltpu.VMEM((CH,), jnp.float32),
    ],
)(_sc_head_body)


def kernel(x, edge_index, edge_label_index, W_t, b_t, W1, b1, W2, b2):
    h = _tc_transform(x, W_t.T, b_t[None, :])

    hsplit = jnp.stack([
        jnp.pad(h[:, :F], ((0, N_PAD - N), (0, 0))),
        jnp.pad(h[:, F:], ((0, N_PAD - N), (0, 0))),
    ])
    row3 = jnp.pad(edge_index[0], (0, E_PAD - E)).reshape(NS, NCH, CH)
    col3 = jnp.pad(edge_index[1], (0, E_PAD - E)).reshape(NS, NCH, CH)

    out_mp = _sc_mp(hsplit, row3, col3)
    hid = jnp.concatenate([out_mp[0, :N], out_mp[1, :N]], axis=1)

    a_t, b_tbl = _tc_proj(hid, W1[:, :D].T, W1[:, D:].T, b1[None, :])

    li3 = jnp.pad(edge_label_index[0],
                  (0, EL_PAD - EL)).reshape(NS * NC, NLCH, CH)
    lj3 = jnp.pad(edge_label_index[1],
                  (0, EL_PAD - EL)).reshape(NS * NC, NLCH, CH)
    wvec = jnp.concatenate([W2[0], b2, jnp.zeros((L - 1,), jnp.float32)])

    pred = _sc_head(a_t, b_tbl, li3, lj3, wvec)
    return pred[:EL, None]


# speedup vs baseline: 10.0173x; 1.5619x over previous
import functools

import jax
import jax.numpy as jnp
from jax import lax
from jax.experimental import pallas as pl
from jax.experimental.pallas import tpu as pltpu
from jax.experimental.pallas import tpu_sc as plsc

N = 10000
E = 320000
EL = 320000
D = 128
F = 64
NC, NS, L = 2, 16, 16
ROWS_PT = 640
RC = 32
N_PAD = NS * ROWS_PT
TRASH = N
CH = 128
G = 8
EPT = 20480
NCH = EPT // CH
NG = NCH // G
E_PAD = EPT * NS
ELPT = 10240
NLCH = ELPT // CH
EL_PAD = ELPT * NS * NC

_mesh = plsc.VectorSubcoreMesh(
    core_axis_name="c", subcore_axis_name="s", num_cores=NC, num_subcores=NS)


def _tc_transform_body(x_ref, w_ref, b_ref, h_ref):
    h_ref[...] = jnp.maximum(
        jnp.dot(x_ref[...], w_ref[...], preferred_element_type=jnp.float32)
        + b_ref[...], 0.0)


def _tc_transform(x, wt_t, b_t):
    return pl.pallas_call(
        _tc_transform_body,
        grid=(10,),
        in_specs=[
            pl.BlockSpec((1000, D), lambda i: (i, 0)),
            pl.BlockSpec((D, D), lambda i: (0, 0)),
            pl.BlockSpec((1, D), lambda i: (0, 0)),
        ],
        out_specs=pl.BlockSpec((1000, D), lambda i: (i, 0)),
        out_shape=jax.ShapeDtypeStruct((N, D), jnp.float32),
    )(x, wt_t, b_t)


def _tc_proj_body(h_ref, wa_ref, wb_ref, b1_ref, a_ref, b_ref):
    hid = h_ref[...]
    n2 = jnp.sum(hid * hid, axis=1, keepdims=True)
    hn = hid / jnp.maximum(jnp.sqrt(n2), 1e-12)
    a_ref[...] = jnp.dot(hn, wa_ref[...],
                         preferred_element_type=jnp.float32) + b1_ref[...]
    b_ref[...] = jnp.dot(hn, wb_ref[...], preferred_element_type=jnp.float32)


def _tc_proj(hid, w1a_t, w1b_t, b1):
    return pl.pallas_call(
        _tc_proj_body,
        grid=(10,),
        in_specs=[
            pl.BlockSpec((1000, D), lambda i: (i, 0)),
            pl.BlockSpec((D, D), lambda i: (0, 0)),
            pl.BlockSpec((D, D), lambda i: (0, 0)),
            pl.BlockSpec((1, D), lambda i: (0, 0)),
        ],
        out_specs=[
            pl.BlockSpec((1000, D), lambda i: (i, 0)),
            pl.BlockSpec((1000, D), lambda i: (i, 0)),
        ],
        out_shape=[
            jax.ShapeDtypeStruct((N, D), jnp.float32),
            jax.ShapeDtypeStruct((N, D), jnp.float32),
        ],
    )(hid, w1a_t, w1b_t, b1)


def _sc_mp_body(h_hbm, row_hbm, col_hbm, out_hbm,
                table, acc, deg,
                rowi, coli, gbuf, hbuf, abuf,
                degv, disv, ewv, isem, gsem, ssem):
    cid = lax.axis_index("c")
    sid = lax.axis_index("s")
    rbase = sid * ROWS_PT

    zero16 = jnp.zeros((L,), jnp.float32)

    def zfill_body(r, _):
        for c4 in range(F // L):
            hbuf[r, pl.ds(c4 * L, L)] = zero16
        return _
    lax.fori_loop(0, RC, zfill_body, None)

    def zdeg_body(i, _):
        degv[pl.ds(i * L, L)] = zero16
        return _
    lax.fori_loop(0, ROWS_PT // L, zdeg_body, None)

    def zacc_body(k, _):
        pltpu.sync_copy(hbuf, acc.at[pl.ds(rbase + k * RC, RC), :])
        return _
    lax.fori_loop(0, ROWS_PT // RC, zacc_body, None)
    pltpu.sync_copy(degv, deg.at[pl.ds(rbase, ROWS_PT)])

    def load_slab(jj, p):
        pltpu.async_copy(row_hbm.at[sid, pl.ds(jj * G, G), :],
                         rowi.at[p], isem)
        pltpu.async_copy(col_hbm.at[sid, pl.ds(jj * G, G), :],
                         coli.at[p], isem)

    def wait_slab(jj, p):
        pltpu.make_async_copy(row_hbm.at[sid, pl.ds(jj * G, G), :],
                              rowi.at[p], isem).wait()
        pltpu.make_async_copy(col_hbm.at[sid, pl.ds(jj * G, G), :],
                              coli.at[p], isem).wait()

    load_slab(0, 0)

    def hist_body(jj, _):
        p = lax.rem(jj, 2)
        wait_slab(jj, p)

        @pl.when(jj + 1 < NG)
        def _n():
            load_slab(jj + 1, 1 - p)
        for j in range(G):
            for i in range(CH // L):
                r = rowi[p, j, pl.ds(i * L, L)]
                c = coli[p, j, pl.ds(i * L, L)]
                ewv[j, pl.ds(i * L, L)] = jnp.where(r == c, 0.0, 1.0)
            pltpu.async_copy(ewv.at[j], deg.at[rowi.at[p, j]], ssem,
                             add=True)
        for j in range(G):
            pltpu.make_async_copy(ewv.at[j], deg.at[rowi.at[p, j]],
                                  ssem).wait()
        return _
    lax.fori_loop(0, NG, hist_body, None)
    plsc.subcore_barrier()

    pltpu.sync_copy(deg.at[pl.ds(rbase, ROWS_PT)], degv)

    def dis_body(i, _):
        d = degv[pl.ds(i * L, L)] + 1.0
        z = 0.5 * (d + 1.0)
        for _n in range(15):
            z = 0.5 * (z + d / z)
        disv[pl.ds(i * L, L)] = 1.0 / z
        return _
    lax.fori_loop(0, ROWS_PT // L, dis_body, None)

    def scale_h_body(k, _):
        r0 = rbase + k * RC
        pltpu.sync_copy(h_hbm.at[cid, pl.ds(r0, RC), :], hbuf)

        def grp_body(g, _2):
            dis16 = disv[pl.ds(k * RC + g * L, L)]
            for r16 in range(L):
                r = g * L + r16
                s = dis16[r16]
                for c4 in range(F // L):
                    hbuf[r, pl.ds(c4 * L, L)] = hbuf[r, pl.ds(c4 * L, L)] * s
            return _2
        lax.fori_loop(0, RC // L, grp_body, None)
        pltpu.sync_copy(hbuf, table.at[pl.ds(r0, RC), :])
        return _
    lax.fori_loop(0, ROWS_PT // RC, scale_h_body, None)
    plsc.subcore_barrier()

    def edge_pass():
        def fire_g(p, j, q):
            pltpu.async_copy(table.at[rowi.at[p, j]], gbuf.at[q], gsem)

        def wait_g(p, j, q):
            pltpu.make_async_copy(table.at[rowi.at[p, j]], gbuf.at[q],
                                  gsem).wait()

        def fire_s(p, j, q):
            pltpu.async_copy(gbuf.at[q], acc.at[coli.at[p, j]], ssem,
                             add=True)

        def wait_s(p, j, q):
            pltpu.make_async_copy(gbuf.at[q], acc.at[coli.at[p, j]],
                                  ssem).wait()

        load_slab(0, 0)

        def slab_body(jj, _):
            p = lax.rem(jj, 2)
            wait_slab(jj, p)

            @pl.when(jj + 1 < NG)
            def _n():
                load_slab(jj + 1, 1 - p)
            for j in range(G):
                for i in range(CH // L):
                    r = rowi[p, j, pl.ds(i * L, L)]
                    c = coli[p, j, pl.ds(i * L, L)]
                    coli[p, j, pl.ds(i * L, L)] = jnp.where(
                        r == c, TRASH, c)
            fire_g(p, 0, 0)
            for j in range(G):
                wait_g(p, j, j % 2)
                if j + 1 < G:
                    if j >= 1:
                        wait_s(p, j - 1, (j - 1) % 2)
                    fire_g(p, j + 1, (j + 1) % 2)
                fire_s(p, j, j % 2)
            wait_s(p, G - 2, (G - 2) % 2)
            wait_s(p, G - 1, (G - 1) % 2)
            return _
        lax.fori_loop(0, NG, slab_body, None)
        plsc.subcore_barrier()

    edge_pass()

    def mid_body(k, _):
        r0 = rbase + k * RC
        pltpu.sync_copy(acc.at[pl.ds(r0, RC), :], abuf)
        pltpu.sync_copy(table.at[pl.ds(r0, RC), :], hbuf)

        def grp_body(g, _2):
            dis16 = disv[pl.ds(k * RC + g * L, L)]
            for r16 in range(L):
                r = g * L + r16
                s = dis16[r16]
                for c4 in range(F // L):
                    sl = pl.ds(c4 * L, L)
                    s1 = (abuf[r, sl] + hbuf[r, sl]) * s
                    hbuf[r, sl] = s1 * s
                    abuf[r, sl] = zero16
            return _2
        lax.fori_loop(0, RC // L, grp_body, None)
        pltpu.sync_copy(hbuf, table.at[pl.ds(r0, RC), :])
        pltpu.sync_copy(abuf, acc.at[pl.ds(r0, RC), :])
        return _
    lax.fori_loop(0, ROWS_PT // RC, mid_body, None)
    plsc.subcore_barrier()

    edge_pass()

    def fin_body(k, _):
        r0 = rbase + k * RC
        pltpu.sync_copy(acc.at[pl.ds(r0, RC), :], abuf)
        pltpu.sync_copy(table.at[pl.ds(r0, RC), :], hbuf)
        h2buf = gbuf.at[0, pl.ds(0, RC), :]
        pltpu.sync_copy(h_hbm.at[cid, pl.ds(r0, RC), :], h2buf)

        def grp_body(g, _2):
            dis16 = disv[pl.ds(k * RC + g * L, L)]
            inv16 = 1.0 / dis16
            for r16 in range(L):
                r = g * L + r16
                s = dis16[r16]
                inv = inv16[r16]
                for c4 in range(F // L):
                    sl = pl.ds(c4 * L, L)
                    t = hbuf[r, sl]
                    s2 = (abuf[r, sl] + t) * s
                    hbuf[r, sl] = h2buf[r, sl] + t * inv + s2
            return _2
        lax.fori_loop(0, RC // L, grp_body, None)
        pltpu.sync_copy(hbuf, out_hbm.at[cid, pl.ds(r0, RC), :])
        return _
    lax.fori_loop(0, ROWS_PT // RC, fin_body, None)


_sc_mp = functools.partial(
    pl.kernel,
    out_type=jax.ShapeDtypeStruct((NC, N_PAD, F), jnp.float32),
    mesh=_mesh,
    scratch_types=[
        pltpu.VMEM_SHARED((N_PAD, F), jnp.float32),
        pltpu.VMEM_SHARED((N_PAD, F), jnp.float32),
        pltpu.VMEM_SHARED((N_PAD,), jnp.float32),
        pltpu.VMEM((2, G, CH), jnp.int32),
        pltpu.VMEM((2, G, CH), jnp.int32),
        pltpu.VMEM((2, CH, F), jnp.float32),
        pltpu.VMEM((RC, F), jnp.float32),
        pltpu.VMEM((RC, F), jnp.float32),
        pltpu.VMEM((ROWS_PT,), jnp.float32),
        pltpu.VMEM((ROWS_PT,), jnp.float32),
        pltpu.VMEM((G, CH), jnp.float32),
        pltpu.SemaphoreType.DMA,
        pltpu.SemaphoreType.DMA,
        pltpu.SemaphoreType.DMA,
    ],
)(_sc_mp_body)


def _sc_head_body(a_hbm, b_hbm, li_hbm, lj_hbm, w_hbm, out_hbm,
                  liv, ljv, abuf, bbuf, wv, obuf, gsem):
    cid = lax.axis_index("c")
    sid = lax.axis_index("s")
    wid = sid * NC + cid
    pltpu.sync_copy(li_hbm.at[wid], liv)
    pltpu.sync_copy(lj_hbm.at[wid], ljv)
    pltpu.sync_copy(w_hbm, wv)
    w2c = [wv[pl.ds(c * L, L)] for c in range(D // L)]
    b2s = wv[pl.ds(D, L)][0]
    lane = lax.iota(jnp.int32, L)
    onehot = [jnp.where(lane == r, 1.0, 0.0) for r in range(L)]

    def fire(j, q):
        pltpu.async_copy(a_hbm.at[liv.at[j]], abuf.at[q], gsem)
        pltpu.async_copy(b_hbm.at[ljv.at[j]], bbuf.at[q], gsem)

    def drain(j, q):
        pltpu.make_async_copy(a_hbm.at[liv.at[j]], abuf.at[q], gsem).wait()
        pltpu.make_async_copy(b_hbm.at[ljv.at[j]], bbuf.at[q], gsem).wait()

    fire(0, 0)

    def chunk_body(j, _):
        q = lax.rem(j, 2)
        drain(j, q)

        @pl.when(j + 1 < NLCH)
        def _n():
            fire(j + 1, 1 - q)

        def grp_body(g, _2):
            out16 = jnp.zeros((L,), jnp.float32) + b2s
            for e16 in range(L):
                e = g * L + e16
                acc16 = jnp.zeros((L,), jnp.float32)
                for c in range(D // L):
                    sl = pl.ds(c * L, L)
                    z = jnp.maximum(abuf[q, e, sl] + bbuf[q, e, sl], 0.0)
                    acc16 = acc16 + w2c[c] * z
                for sh in (8, 4, 2, 1):
                    acc16 = acc16 + acc16.at[lane ^ sh].get(
                        mode="promise_in_bounds")
                out16 = out16 + acc16 * onehot[e16]
            obuf[pl.ds(j * CH + g * L, L)] = out16
            return _2
        lax.fori_loop(0, CH // L, grp_body, None)
        return _
    lax.fori_loop(0, NLCH, chunk_body, None)
    pltpu.sync_copy(obuf, out_hbm.at[pl.ds(wid * ELPT, ELPT)])


_sc_head = functools.partial(
    pl.kernel,
    out_type=jax.ShapeDtypeStruct((EL_PAD,), jnp.float32),
    mesh=_mesh,
    scratch_types=[
        pltpu.VMEM((NLCH, CH), jnp.int32),
        pltpu.VMEM((NLCH, CH), jnp.int32),
        pltpu.VMEM((2, CH, D), jnp.float32),
        pltpu.VMEM((2, CH, D), jnp.float32),
        pltpu.VMEM((D + L,), jnp.float32),
        pltpu.VMEM((ELPT,), jnp.float32),
        pltpu.SemaphoreType.DMA,
    ],
)(_sc_head_body)


def kernel(x, edge_index, edge_label_index, W_t, b_t, W1, b1, W2, b2):
    h = _tc_transform(x, W_t.T, b_t[None, :])

    hsplit = jnp.stack([
        jnp.pad(h[:, :F], ((0, N_PAD - N), (0, 0))),
        jnp.pad(h[:, F:], ((0, N_PAD - N), (0, 0))),
    ])
    row3 = jnp.pad(edge_index[0], (0, E_PAD - E)).reshape(NS, NCH, CH)
    col3 = jnp.pad(edge_index[1], (0, E_PAD - E)).reshape(NS, NCH, CH)

    out_mp = _sc_mp(hsplit, row3, col3)
    hid = jnp.concatenate([out_mp[0, :N], out_mp[1, :N]], axis=1)

    a_t, b_tbl = _tc_proj(hid, W1[:, :D].T, W1[:, D:].T, b1[None, :])

    li3 = jnp.pad(edge_label_index[0],
                  (0, EL_PAD - EL)).reshape(NS * NC, NLCH, CH)
    lj3 = jnp.pad(edge_label_index[1],
                  (0, EL_PAD - EL)).reshape(NS * NC, NLCH, CH)
    wvec = jnp.concatenate([W2[0], b2, jnp.zeros((L - 1,), jnp.float32)])

    pred = _sc_head(a_t, b_tbl, li3, lj3, wvec)
    return pred[:EL, None]


# speedup vs baseline: 11.7248x; 1.1705x over previous
import functools

import jax
import jax.numpy as jnp
from jax import lax
from jax.experimental import pallas as pl
from jax.experimental.pallas import tpu as pltpu
from jax.experimental.pallas import tpu_sc as plsc

N = 10000
E = 320000
EL = 320000
D = 128
F = 64
NC, NS, L = 2, 16, 16
ROWS_PT = 640
RC = 32
N_PAD = NS * ROWS_PT
TRASH = N
CH = 128
G = 8
EPT = 20480
NCH = EPT // CH
NG = NCH // G
E_PAD = EPT * NS
ELPT = 10240
NLCH = ELPT // CH
EL_PAD = ELPT * NS * NC

_mesh = plsc.VectorSubcoreMesh(
    core_axis_name="c", subcore_axis_name="s", num_cores=NC, num_subcores=NS)


def _tc_transform_body(x_ref, w_ref, b_ref, h_ref):
    h_ref[...] = jnp.maximum(
        jnp.dot(x_ref[...], w_ref[...], preferred_element_type=jnp.float32)
        + b_ref[...], 0.0)


def _tc_transform(x, wt_t, b_t):
    return pl.pallas_call(
        _tc_transform_body,
        grid=(10,),
        in_specs=[
            pl.BlockSpec((1000, D), lambda i: (i, 0)),
            pl.BlockSpec((D, D), lambda i: (0, 0)),
            pl.BlockSpec((1, D), lambda i: (0, 0)),
        ],
        out_specs=pl.BlockSpec((1000, D), lambda i: (i, 0)),
        out_shape=jax.ShapeDtypeStruct((N, D), jnp.float32),
    )(x, wt_t, b_t)


def _tc_proj_body(h_ref, wa_ref, wb_ref, b1_ref, a_ref, b_ref):
    hid = jnp.concatenate([h_ref[0], h_ref[1]], axis=1)
    n2 = jnp.sum(hid * hid, axis=1, keepdims=True)
    hn = hid / jnp.maximum(jnp.sqrt(n2), 1e-12)
    a_ref[...] = jnp.dot(hn, wa_ref[...],
                         preferred_element_type=jnp.float32) + b1_ref[...]
    b_ref[...] = jnp.dot(hn, wb_ref[...], preferred_element_type=jnp.float32)


def _tc_proj(out_mp, w1a_t, w1b_t, b1):
    return pl.pallas_call(
        _tc_proj_body,
        grid=(16,),
        in_specs=[
            pl.BlockSpec((2, ROWS_PT, F), lambda i: (0, i, 0)),
            pl.BlockSpec((D, D), lambda i: (0, 0)),
            pl.BlockSpec((D, D), lambda i: (0, 0)),
            pl.BlockSpec((1, D), lambda i: (0, 0)),
        ],
        out_specs=[
            pl.BlockSpec((ROWS_PT, D), lambda i: (i, 0)),
            pl.BlockSpec((ROWS_PT, D), lambda i: (i, 0)),
        ],
        out_shape=[
            jax.ShapeDtypeStruct((N_PAD, D), jnp.float32),
            jax.ShapeDtypeStruct((N_PAD, D), jnp.float32),
        ],
    )(out_mp, w1a_t, w1b_t, b1)


def _sc_mp_body(h_hbm, row_hbm, col_hbm, out_hbm,
                table, acc, deg,
                rowi, coli, gbuf, hbuf, abuf,
                degv, disv, ewv, isem, gsem, ssem):
    cid = lax.axis_index("c")
    sid = lax.axis_index("s")
    rbase = sid * ROWS_PT

    zero16 = jnp.zeros((L,), jnp.float32)

    def zfill_body(r, _):
        for c4 in range(F // L):
            hbuf[r, pl.ds(c4 * L, L)] = zero16
        return _
    lax.fori_loop(0, RC, zfill_body, None)

    def zdeg_body(i, _):
        degv[pl.ds(i * L, L)] = zero16
        return _
    lax.fori_loop(0, ROWS_PT // L, zdeg_body, None)

    def zacc_body(k, _):
        pltpu.sync_copy(hbuf, acc.at[pl.ds(rbase + k * RC, RC), :])
        return _
    lax.fori_loop(0, ROWS_PT // RC, zacc_body, None)
    pltpu.sync_copy(degv, deg.at[pl.ds(rbase, ROWS_PT)])

    def load_slab(jj, p):
        pltpu.async_copy(row_hbm.at[sid, pl.ds(jj * G, G), :],
                         rowi.at[p], isem)
        pltpu.async_copy(col_hbm.at[sid, pl.ds(jj * G, G), :],
                         coli.at[p], isem)

    def wait_slab(jj, p):
        pltpu.make_async_copy(row_hbm.at[sid, pl.ds(jj * G, G), :],
                              rowi.at[p], isem).wait()
        pltpu.make_async_copy(col_hbm.at[sid, pl.ds(jj * G, G), :],
                              coli.at[p], isem).wait()

    load_slab(0, 0)

    def hist_body(jj, _):
        p = lax.rem(jj, 2)
        wait_slab(jj, p)

        @pl.when(jj + 1 < NG)
        def _n():
            load_slab(jj + 1, 1 - p)
        for j in range(G):
            for i in range(CH // L):
                r = rowi[p, j, pl.ds(i * L, L)]
                c = coli[p, j, pl.ds(i * L, L)]
                ewv[j, pl.ds(i * L, L)] = jnp.where(r == c, 0.0, 1.0)
            pltpu.async_copy(ewv.at[j], deg.at[rowi.at[p, j]], ssem,
                             add=True)
        for j in range(G):
            pltpu.make_async_copy(ewv.at[j], deg.at[rowi.at[p, j]],
                                  ssem).wait()
        return _
    lax.fori_loop(0, NG, hist_body, None)
    plsc.subcore_barrier()

    pltpu.sync_copy(deg.at[pl.ds(rbase, ROWS_PT)], degv)

    def dis_body(i, _):
        d = degv[pl.ds(i * L, L)] + 1.0
        z = 0.5 * (d + 1.0)
        for _n in range(15):
            z = 0.5 * (z + d / z)
        disv[pl.ds(i * L, L)] = 1.0 / z
        return _
    lax.fori_loop(0, ROWS_PT // L, dis_body, None)

    def scale_h_body(k, _):
        r0 = rbase + k * RC
        pltpu.sync_copy(h_hbm.at[cid, pl.ds(r0, RC), :], hbuf)

        def grp_body(g, _2):
            dis16 = disv[pl.ds(k * RC + g * L, L)]
            for r16 in range(L):
                r = g * L + r16
                s = dis16[r16]
                for c4 in range(F // L):
                    hbuf[r, pl.ds(c4 * L, L)] = hbuf[r, pl.ds(c4 * L, L)] * s
            return _2
        lax.fori_loop(0, RC // L, grp_body, None)
        pltpu.sync_copy(hbuf, table.at[pl.ds(r0, RC), :])
        return _
    lax.fori_loop(0, ROWS_PT // RC, scale_h_body, None)
    plsc.subcore_barrier()

    def edge_pass():
        def fire_g(p, j, q):
            pltpu.async_copy(table.at[rowi.at[p, j]], gbuf.at[q], gsem)

        def wait_g(p, j, q):
            pltpu.make_async_copy(table.at[rowi.at[p, j]], gbuf.at[q],
                                  gsem).wait()

        def fire_s(p, j, q):
            pltpu.async_copy(gbuf.at[q], acc.at[coli.at[p, j]], ssem,
                             add=True)

        def wait_s(p, j, q):
            pltpu.make_async_copy(gbuf.at[q], acc.at[coli.at[p, j]],
                                  ssem).wait()

        load_slab(0, 0)

        def slab_body(jj, _):
            p = lax.rem(jj, 2)
            wait_slab(jj, p)

            @pl.when(jj + 1 < NG)
            def _n():
                load_slab(jj + 1, 1 - p)
            for j in range(G):
                for i in range(CH // L):
                    r = rowi[p, j, pl.ds(i * L, L)]
                    c = coli[p, j, pl.ds(i * L, L)]
                    coli[p, j, pl.ds(i * L, L)] = jnp.where(
                        r == c, TRASH, c)
            fire_g(p, 0, 0)
            for j in range(G):
                wait_g(p, j, j % 2)
                if j + 1 < G:
                    if j >= 1:
                        wait_s(p, j - 1, (j - 1) % 2)
                    fire_g(p, j + 1, (j + 1) % 2)
                fire_s(p, j, j % 2)
            wait_s(p, G - 2, (G - 2) % 2)
            wait_s(p, G - 1, (G - 1) % 2)
            return _
        lax.fori_loop(0, NG, slab_body, None)
        plsc.subcore_barrier()

    edge_pass()

    def mid_body(k, _):
        r0 = rbase + k * RC
        pltpu.sync_copy(acc.at[pl.ds(r0, RC), :], abuf)
        pltpu.sync_copy(table.at[pl.ds(r0, RC), :], hbuf)

        def grp_body(g, _2):
            dis16 = disv[pl.ds(k * RC + g * L, L)]
            for r16 in range(L):
                r = g * L + r16
                s = dis16[r16]
                for c4 in range(F // L):
                    sl = pl.ds(c4 * L, L)
                    s1 = (abuf[r, sl] + hbuf[r, sl]) * s
                    hbuf[r, sl] = s1 * s
                    abuf[r, sl] = zero16
            return _2
        lax.fori_loop(0, RC // L, grp_body, None)
        pltpu.sync_copy(hbuf, table.at[pl.ds(r0, RC), :])
        pltpu.sync_copy(abuf, acc.at[pl.ds(r0, RC), :])
        return _
    lax.fori_loop(0, ROWS_PT // RC, mid_body, None)
    plsc.subcore_barrier()

    edge_pass()

    def fin_body(k, _):
        r0 = rbase + k * RC
        pltpu.sync_copy(acc.at[pl.ds(r0, RC), :], abuf)
        pltpu.sync_copy(table.at[pl.ds(r0, RC), :], hbuf)
        h2buf = gbuf.at[0, pl.ds(0, RC), :]
        pltpu.sync_copy(h_hbm.at[cid, pl.ds(r0, RC), :], h2buf)

        def grp_body(g, _2):
            dis16 = disv[pl.ds(k * RC + g * L, L)]
            inv16 = 1.0 / dis16
            for r16 in range(L):
                r = g * L + r16
                s = dis16[r16]
                inv = inv16[r16]
                for c4 in range(F // L):
                    sl = pl.ds(c4 * L, L)
                    t = hbuf[r, sl]
                    s2 = (abuf[r, sl] + t) * s
                    hbuf[r, sl] = h2buf[r, sl] + t * inv + s2
            return _2
        lax.fori_loop(0, RC // L, grp_body, None)
        pltpu.sync_copy(hbuf, out_hbm.at[cid, pl.ds(r0, RC), :])
        return _
    lax.fori_loop(0, ROWS_PT // RC, fin_body, None)


_sc_mp = functools.partial(
    pl.kernel,
    out_type=jax.ShapeDtypeStruct((NC, N_PAD, F), jnp.float32),
    mesh=_mesh,
    scratch_types=[
        pltpu.VMEM_SHARED((N_PAD, F), jnp.float32),
        pltpu.VMEM_SHARED((N_PAD, F), jnp.float32),
        pltpu.VMEM_SHARED((N_PAD,), jnp.float32),
        pltpu.VMEM((2, G, CH), jnp.int32),
        pltpu.VMEM((2, G, CH), jnp.int32),
        pltpu.VMEM((2, CH, F), jnp.float32),
        pltpu.VMEM((RC, F), jnp.float32),
        pltpu.VMEM((RC, F), jnp.float32),
        pltpu.VMEM((ROWS_PT,), jnp.float32),
        pltpu.VMEM((ROWS_PT,), jnp.float32),
        pltpu.VMEM((G, CH), jnp.float32),
        pltpu.SemaphoreType.DMA,
        pltpu.SemaphoreType.DMA,
        pltpu.SemaphoreType.DMA,
    ],
)(_sc_mp_body)


HCH = 64
HPT = EL_PAD // NS
NHC = HPT // HCH
NHG = NHC // G
NSTG = N_PAD // NS


def _sc_head_body(ab_hbm, li_hbm, lj_hbm, w_hbm, out_hbm,
                  asp, bsp, liv, ljv, abuf, bbuf, wv, obuf,
                  isem, gsem, osem):
    cid = lax.axis_index("c")
    sid = lax.axis_index("s")
    pltpu.sync_copy(w_hbm.at[cid], wv)
    w2c = [wv[pl.ds(c * L, L)] for c in range(F // L)]
    b2s = wv[pl.ds(F, L)][0]
    lane = lax.iota(jnp.int32, L)
    onehot = [jnp.where(lane == r, 1.0, 0.0) for r in range(L)]

    def stg_body(k, _):
        r0 = sid * NSTG + k * HCH
        pltpu.sync_copy(ab_hbm.at[0, cid, pl.ds(r0, HCH), :], abuf.at[0])
        pltpu.sync_copy(abuf.at[0], asp.at[pl.ds(r0, HCH), :])
        pltpu.sync_copy(ab_hbm.at[1, cid, pl.ds(r0, HCH), :], bbuf.at[0])
        pltpu.sync_copy(bbuf.at[0], bsp.at[pl.ds(r0, HCH), :])
        return _
    lax.fori_loop(0, NSTG // HCH, stg_body, None)
    plsc.subcore_barrier()

    def load_slab(jj, p):
        pltpu.async_copy(li_hbm.at[sid, pl.ds(jj * G, G), :],
                         liv.at[p], isem)
        pltpu.async_copy(lj_hbm.at[sid, pl.ds(jj * G, G), :],
                         ljv.at[p], isem)

    def wait_slab(jj, p):
        pltpu.make_async_copy(li_hbm.at[sid, pl.ds(jj * G, G), :],
                              liv.at[p], isem).wait()
        pltpu.make_async_copy(lj_hbm.at[sid, pl.ds(jj * G, G), :],
                              ljv.at[p], isem).wait()

    def fire(p, j, q):
        pltpu.async_copy(asp.at[liv.at[p, j]], abuf.at[q], gsem)
        pltpu.async_copy(bsp.at[ljv.at[p, j]], bbuf.at[q], gsem)

    def drain(p, j, q):
        pltpu.make_async_copy(asp.at[liv.at[p, j]], abuf.at[q], gsem).wait()
        pltpu.make_async_copy(bsp.at[ljv.at[p, j]], bbuf.at[q], gsem).wait()

    load_slab(0, 0)

    def slab_body(jj, _):
        p = lax.rem(jj, 2)
        wait_slab(jj, p)

        @pl.when(jj + 1 < NHG)
        def _n():
            load_slab(jj + 1, 1 - p)
        fire(p, 0, 0)
        for j in range(G):
            q = j % 2
            drain(p, j, q)
            if j + 1 < G:
                fire(p, j + 1, 1 - q)

            def grp_body(g, _2, j=j, q=q, p=p):
                out16 = jnp.zeros((L,), jnp.float32) + b2s
                for e16 in range(L):
                    e = g * L + e16
                    acc16 = jnp.zeros((L,), jnp.float32)
                    for c in range(F // L):
                        sl = pl.ds(c * L, L)
                        z = jnp.maximum(abuf[q, e, sl] + bbuf[q, e, sl], 0.0)
                        acc16 = acc16 + w2c[c] * z
                    for sh in (8, 4, 2, 1):
                        acc16 = acc16 + acc16.at[lane ^ sh].get(
                            mode="promise_in_bounds")
                    out16 = out16 + acc16 * onehot[e16]
                obuf[p, pl.ds(j * HCH + g * L, L)] = out16
                return _2
            lax.fori_loop(0, HCH // L, grp_body, None)

        @pl.when(jj >= 1)
        def _w():
            pltpu.make_async_copy(
                obuf.at[1 - p],
                out_hbm.at[cid, pl.ds(sid * HPT + (jj - 1) * G * HCH,
                                      G * HCH)],
                osem).wait()
        pltpu.async_copy(
            obuf.at[p],
            out_hbm.at[cid, pl.ds(sid * HPT + jj * G * HCH, G * HCH)], osem)
        return _
    lax.fori_loop(0, NHG, slab_body, None)
    pltpu.make_async_copy(
        obuf.at[lax.rem(NHG - 1, 2)],
        out_hbm.at[cid, pl.ds(sid * HPT + (NHG - 1) * G * HCH, G * HCH)],
        osem).wait()


_sc_head = functools.partial(
    pl.kernel,
    out_type=jax.ShapeDtypeStruct((NC, EL_PAD), jnp.float32),
    mesh=_mesh,
    scratch_types=[
        pltpu.VMEM_SHARED((N_PAD, F), jnp.float32),
        pltpu.VMEM_SHARED((N_PAD, F), jnp.float32),
        pltpu.VMEM((2, G, HCH), jnp.int32),
        pltpu.VMEM((2, G, HCH), jnp.int32),
        pltpu.VMEM((2, HCH, F), jnp.float32),
        pltpu.VMEM((2, HCH, F), jnp.float32),
        pltpu.VMEM((F + L,), jnp.float32),
        pltpu.VMEM((2, G * HCH), jnp.float32),
        pltpu.SemaphoreType.DMA,
        pltpu.SemaphoreType.DMA,
        pltpu.SemaphoreType.DMA,
    ],
)(_sc_head_body)


def _tc_comb_body(p_ref, o_ref):
    o_ref[...] = p_ref[0] + p_ref[1]


def _tc_comb(parts):
    return pl.pallas_call(
        _tc_comb_body,
        grid=(10,),
        in_specs=[pl.BlockSpec((2, 32, 1024), lambda i: (0, i, 0))],
        out_specs=pl.BlockSpec((32, 1024), lambda i: (i, 0)),
        out_shape=jax.ShapeDtypeStruct((EL_PAD // 1024, 1024), jnp.float32),
    )(parts)


def kernel(x, edge_index, edge_label_index, W_t, b_t, W1, b1, W2, b2):
    h = _tc_transform(x, W_t.T, b_t[None, :])
    hsplit = jnp.stack([
        jnp.pad(h[:, :F], ((0, N_PAD - N), (0, 0))),
        jnp.pad(h[:, F:], ((0, N_PAD - N), (0, 0))),
    ])

    row3 = jnp.pad(edge_index[0], (0, E_PAD - E)).reshape(NS, NCH, CH)
    col3 = jnp.pad(edge_index[1], (0, E_PAD - E)).reshape(NS, NCH, CH)

    out_mp = _sc_mp(hsplit, row3, col3)

    a_t, b_t2 = _tc_proj(out_mp, W1[:, :D].T, W1[:, D:].T, b1[None, :])
    ab = jnp.stack([
        jnp.stack([a_t[:, :F], a_t[:, F:]]),
        jnp.stack([b_t2[:, :F], b_t2[:, F:]]),
    ])

    li3 = jnp.pad(edge_label_index[0],
                  (0, EL_PAD - EL)).reshape(NS, NHC, HCH)
    lj3 = jnp.pad(edge_label_index[1],
                  (0, EL_PAD - EL)).reshape(NS, NHC, HCH)

    w2 = W2[0]
    tail0 = jnp.concatenate([b2, jnp.zeros((L - 1,), jnp.float32)])
    tail1 = jnp.zeros((L,), jnp.float32)
    wvec = jnp.stack([jnp.concatenate([w2[:F], tail0]),
                      jnp.concatenate([w2[F:], tail1])])

    parts = _sc_head(ab, li3, lj3, wvec)
    pred = _tc_comb(parts.reshape(NC, EL_PAD // 1024, 1024))
    return pred.reshape(EL_PAD)[:EL, None]
